# baseline (speedup 1.0000x reference)
import functools

import numpy as np
import jax
import jax.numpy as jnp
from jax import lax
from jax.experimental import pallas as pl
from jax.experimental.pallas import tpu as pltpu
from jax.experimental.pallas import tpu_sc as plsc

_B = 16
_n = 2048
_m = 512
_K = 64
_H = 64
_OUT = 64
_F = 8
_R2 = np.float32(0.15 * 0.15)
_R2_BITS = int(np.array(0.15 * 0.15, np.float32).view(np.int32))

_NC = 2
_NS = 16
_NW = _NC * _NS
_HM = (_B * _m) // _NW


def _fps_body(px_ref, py_ref, pz_ref, idx_ref):
    px = px_ref[...]
    py = py_ref[...]
    pz = pz_ref[...]
    iota = lax.broadcasted_iota(jnp.int32, (_B, _n), 1)
    iota_m = lax.broadcasted_iota(jnp.int32, (_B, _m), 1)
    cx = px[:, 0:1]
    cy = py[:, 0:1]
    cz = pz[:, 0:1]
    dx = px - cx
    dy = py - cy
    dz = pz - cz
    dists0 = dx * dx + dy * dy + dz * dz
    state0 = (dists0, jnp.zeros((_B, _m), jnp.int32))

    def body(i, state):
        dists, sidx = state
        maxv = jnp.max(dists, axis=1, keepdims=True)
        nxt = jnp.min(jnp.where(dists == maxv, iota, _n), axis=1, keepdims=True)
        oh = iota == nxt
        cx = jnp.sum(jnp.where(oh, px, 0.0), axis=1, keepdims=True)
        cy = jnp.sum(jnp.where(oh, py, 0.0), axis=1, keepdims=True)
        cz = jnp.sum(jnp.where(oh, pz, 0.0), axis=1, keepdims=True)
        sidx = jnp.where(iota_m == i, nxt, sidx)
        dx = px - cx
        dy = py - cy
        dz = pz - cz
        d = dx * dx + dy * dy + dz * dz
        return (jnp.minimum(dists, d), sidx)

    _, sidx = lax.fori_loop(1, _m, body, state0)
    idx_ref[...] = sidx


_fps_call = pl.pallas_call(
    _fps_body,
    out_shape=jax.ShapeDtypeStruct((_B, _m), jnp.int32),
)


def _sc_body(posx_h, posy_h, posz_h, xx_h, xy_h, xz_h,
             seli_h,
             feat_h, selpx_h, selpy_h, selpz_h,
             px_v, py_v, pz_v, xv0, xv1, xv2,
             sci, spb,
             cand_i, fin_i, stage0, stage1, sem0, sem1):
    wid = lax.axis_index("s") * _NC + lax.axis_index("c")
    cloud = wid // 2
    half = wid % 2
    row0 = cloud * _m + half * _HM

    pltpu.sync_copy(posx_h.at[cloud], px_v)
    pltpu.sync_copy(posy_h.at[cloud], py_v)
    pltpu.sync_copy(posz_h.at[cloud], pz_v)
    pltpu.sync_copy(xx_h.at[cloud], xv0)
    pltpu.sync_copy(xy_h.at[cloud], xv1)
    pltpu.sync_copy(xz_h.at[cloud], xv2)
    sl_half = pl.ds(half * _HM, _HM)
    pltpu.sync_copy(seli_h.at[cloud, sl_half], sci)

    for stage in (stage0, stage1):
        for g in range(_K * _F // 16):
            stage[pl.ds(g * 16, 16)] = jnp.zeros((16,), jnp.float32)

    lane16 = lax.iota(jnp.int32, 16)
    r2 = jnp.float32(_R2)

    def cand_d2(c, cnt, cx, cy, cz):
        sl = pl.ds(c * 16, 16)
        iv = cand_i[sl]
        dx = plsc.load_gather(px_v, [iv]) - cx
        dy = plsc.load_gather(py_v, [iv]) - cy
        dz = plsc.load_gather(pz_v, [iv]) - cz
        d2 = dx * dx + dy * dy + dz * dz
        ln = c * 16 + lane16
        return plsc.bitcast(d2, jnp.int32), iv, ln < cnt

    def do_row(r, stage, sem, other_stage, other_sem, first):
        rsplat = jnp.full((16,), r, jnp.int32)
        si = plsc.load_gather(sci, [rsplat])
        cx = plsc.load_gather(px_v, [si])
        cy = plsc.load_gather(py_v, [si])
        cz = plsc.load_gather(pz_v, [si])

        def chunk(c, cur_vec):
            sl = pl.ds(c * 16, 16)
            dx = px_v[sl] - cx
            dy = py_v[sl] - cy
            dz = pz_v[sl] - cz
            d2 = dx * dx + dy * dy + dz * dz
            msk = d2 <= r2
            mi = msk.astype(jnp.int32)
            tgt = cur_vec + plsc.cumsum(mi) - mi
            idxv = c * 16 + lane16
            plsc.store_scatter(cand_i, [tgt], idxv, mask=msk)
            return cur_vec + plsc.all_reduce_population_count(msk)

        cur_vec = lax.fori_loop(0, _n // 16, chunk,
                                jnp.zeros((16,), jnp.int32), unroll=2)
        cnt = cur_vec[0]

        @pl.when(cnt > _K)
        def _fixup():
            nch = (cnt + 15) // 16

            def count_le(v):
                def cc(c, acc):
                    dv, _, inb = cand_d2(c, cnt, cx, cy, cz)
                    mk = (dv <= v) & inb
                    return acc + plsc.all_reduce_population_count(mk)[0]
                return lax.fori_loop(0, nch, cc, jnp.int32(0))

            def bcond(state):
                lo, hi = state
                return lo < hi

            def bstep(state):
                lo, hi = state
                mid = (lo + hi) // 2
                big = count_le(mid) >= _K
                return (jnp.where(big, lo, mid + 1), jnp.where(big, mid, hi))

            vk, _ = lax.while_loop(
                bcond, bstep, (jnp.int32(0), jnp.int32(_R2_BITS)))

            def sel1(c, cur):
                dv, iv, inb = cand_d2(c, cnt, cx, cy, cz)
                mk = (dv < vk) & inb
                plsc.store_compressed(fin_i.at[pl.ds(cur, 16)], iv, mask=mk)
                return cur + plsc.all_reduce_population_count(mk)[0]

            cur = lax.fori_loop(0, nch, sel1, jnp.int32(0))

            def sel2(c, cur):
                dv, iv, inb = cand_d2(c, cnt, cx, cy, cz)
                mk = (dv == vk) & inb
                pref = plsc.cumsum(mk.astype(jnp.int32))
                keep = mk & ((cur + pref) <= _K)
                plsc.store_compressed(fin_i.at[pl.ds(cur, 16)], iv, mask=keep)
                return cur + plsc.all_reduce_population_count(keep)[0]

            lax.fori_loop(0, nch, sel2, cur)
            for g in range(_K // 16):
                sl = pl.ds(g * 16, 16)
                cand_i[sl] = fin_i[sl]

        @pl.when(jnp.logical_not(first))
        def _drain():
            pltpu.make_async_copy(stage, feat_h.at[row0], sem).wait()

        cnt2 = jnp.minimum(cnt, _K)
        for g in range(_K // 16):
            ln = g * 16 + lane16
            cv = cand_i[pl.ds(g * 16, 16)]
            jv = jnp.where(ln < cnt2, cv, si)
            f0 = plsc.load_gather(xv0, [jv])
            f1 = plsc.load_gather(xv1, [jv])
            f2 = plsc.load_gather(xv2, [jv])
            f3 = plsc.load_gather(px_v, [jv]) - cx
            f4 = plsc.load_gather(py_v, [jv]) - cy
            f5 = plsc.load_gather(pz_v, [jv]) - cz
            base = ln * _F
            plsc.store_scatter(stage, [base + 0], f0)
            plsc.store_scatter(stage, [base + 1], f1)
            plsc.store_scatter(stage, [base + 2], f2)
            plsc.store_scatter(stage, [base + 3], f3)
            plsc.store_scatter(stage, [base + 4], f4)
            plsc.store_scatter(stage, [base + 5], f5)
        pltpu.async_copy(stage, feat_h.at[row0 + r], sem)

    def row_pair(rr, carry):
        do_row(rr * 2, stage0, sem0, stage1, sem1, rr == 0)
        do_row(rr * 2 + 1, stage1, sem1, stage0, sem0, rr == 0)
        return carry

    lax.fori_loop(0, _HM // 2, row_pair, jnp.int32(0))
    for g in range(_HM // 16):
        iv = sci[pl.ds(g * 16, 16)]
        spb[pl.ds(g * 16, 16)] = plsc.load_gather(px_v, [iv])
        spb[pl.ds(_HM + g * 16, 16)] = plsc.load_gather(py_v, [iv])
        spb[pl.ds(2 * _HM + g * 16, 16)] = plsc.load_gather(pz_v, [iv])
    for ch, out_h in enumerate((selpx_h, selpy_h, selpz_h)):
        pltpu.sync_copy(spb.at[pl.ds(ch * _HM, _HM)],
                        out_h.at[pl.ds(row0, _HM)])
    pltpu.make_async_copy(stage0, feat_h.at[row0], sem0).wait()
    pltpu.make_async_copy(stage1, feat_h.at[row0], sem1).wait()


@functools.cache
def _make_sc_call():
    mesh = plsc.VectorSubcoreMesh(core_axis_name="c", subcore_axis_name="s")
    return functools.partial(
        pl.kernel,
        mesh=mesh,
        compiler_params=pltpu.CompilerParams(needs_layout_passes=False),
        out_type=[jax.ShapeDtypeStruct((_B * _m, _K * _F), jnp.float32),
                  jax.ShapeDtypeStruct((_B * _m,), jnp.float32),
                  jax.ShapeDtypeStruct((_B * _m,), jnp.float32),
                  jax.ShapeDtypeStruct((_B * _m,), jnp.float32)],
        scratch_types=[
            pltpu.VMEM((_n,), jnp.float32),
            pltpu.VMEM((_n,), jnp.float32),
            pltpu.VMEM((_n,), jnp.float32),
            pltpu.VMEM((_n,), jnp.float32),
            pltpu.VMEM((_n,), jnp.float32),
            pltpu.VMEM((_n,), jnp.float32),
            pltpu.VMEM((_HM,), jnp.int32),
            pltpu.VMEM((3 * _HM,), jnp.float32),
            pltpu.VMEM((_n + 32,), jnp.int32),
            pltpu.VMEM((_K + 16,), jnp.int32),
            pltpu.VMEM((_K * _F,), jnp.float32),
            pltpu.VMEM((_K * _F,), jnp.float32),
            pltpu.SemaphoreType.DMA,
            pltpu.SemaphoreType.DMA,
        ],
    )(_sc_body)


_EB = 8192


def _mlp_body(f_ref, w1_ref, b1_ref, w2_ref, b2_ref, w3_ref, b3_ref, o_ref):
    f = f_ref[...]
    h = jnp.dot(f, w1_ref[...], preferred_element_type=jnp.float32)
    h = jnp.maximum(h + b1_ref[...], 0.0)
    h = jnp.dot(h, w2_ref[...], preferred_element_type=jnp.float32)
    h = jnp.maximum(h + b2_ref[...], 0.0)
    h = jnp.dot(h, w3_ref[...], preferred_element_type=jnp.float32)
    h = jnp.maximum(h + b3_ref[...], 0.0)
    h = h.reshape(_EB // _K, _K, _OUT)
    o_ref[...] = jnp.max(h, axis=1)


_mlp_call = pl.pallas_call(
    _mlp_body,
    grid=(_B * _m * _K // _EB,),
    in_specs=[
        pl.BlockSpec((_EB, _F), lambda i: (i, 0)),
        pl.BlockSpec((_F, _H), lambda i: (0, 0)),
        pl.BlockSpec((1, _H), lambda i: (0, 0)),
        pl.BlockSpec((_H, _H), lambda i: (0, 0)),
        pl.BlockSpec((1, _H), lambda i: (0, 0)),
        pl.BlockSpec((_H, _OUT), lambda i: (0, 0)),
        pl.BlockSpec((1, _OUT), lambda i: (0, 0)),
    ],
    out_specs=pl.BlockSpec((_EB // _K, _OUT), lambda i: (i, 0)),
    out_shape=jax.ShapeDtypeStruct((_B * _m, _OUT), jnp.float32),
)


def kernel(x, pos, batch, W1, b1, W2, b2, W3, b3):
    posb = pos.reshape(_B, _n, 3)
    xb = x.reshape(_B, _n, 3)
    posx = posb[:, :, 0]
    posy = posb[:, :, 1]
    posz = posb[:, :, 2]
    xx = xb[:, :, 0]
    xy = xb[:, :, 1]
    xz = xb[:, :, 2]

    sel_idx = _fps_call(posx, posy, posz)

    feat, spx, spy, spz = _make_sc_call()(posx, posy, posz, xx, xy, xz, sel_idx)
    featr = feat.reshape(_B * _m * _K, _F)

    W1p = jnp.concatenate([W1, jnp.zeros((_F - W1.shape[0], _H), jnp.float32)], axis=0)
    out = _mlp_call(featr, W1p, b1.reshape(1, _H), W2, b2.reshape(1, _H),
                    W3, b3.reshape(1, _OUT))

    sel_pos = jnp.stack([spx, spy, spz], axis=-1)
    sel_batch = jnp.repeat(jnp.arange(_B, dtype=batch.dtype), _m)
    return out, sel_pos, sel_batch


# speedup vs baseline: 1.4100x; 1.4100x over previous
import functools

import numpy as np
import jax
import jax.numpy as jnp
from jax import lax
from jax.experimental import pallas as pl
from jax.experimental.pallas import tpu as pltpu
from jax.experimental.pallas import tpu_sc as plsc

_B = 16
_n = 2048
_m = 512
_K = 64
_H = 64
_OUT = 64
_F = 8
_R2 = np.float32(0.15 * 0.15)
_R2_BITS = int(np.array(0.15 * 0.15, np.float32).view(np.int32))

_NC = 2
_NS = 16
_NW = _NC * _NS
_HM = (_B * _m) // _NW


def _fps_body(px_ref, py_ref, pz_ref, idx_ref):
    px = px_ref[...]
    py = py_ref[...]
    pz = pz_ref[...]
    iota = lax.broadcasted_iota(jnp.int32, (_B, _n), 1)
    iota_m = lax.broadcasted_iota(jnp.int32, (_B, _m), 1)
    cx = px[:, 0:1]
    cy = py[:, 0:1]
    cz = pz[:, 0:1]
    dx = px - cx
    dy = py - cy
    dz = pz - cz
    dists0 = dx * dx + dy * dy + dz * dz
    state0 = (dists0, jnp.zeros((_B, _m), jnp.int32))

    def body(i, state):
        dists, sidx = state
        maxv = jnp.max(dists, axis=1, keepdims=True)
        nxt = jnp.min(jnp.where(dists == maxv, iota, _n), axis=1, keepdims=True)
        oh = iota == nxt
        cx = jnp.sum(jnp.where(oh, px, 0.0), axis=1, keepdims=True)
        cy = jnp.sum(jnp.where(oh, py, 0.0), axis=1, keepdims=True)
        cz = jnp.sum(jnp.where(oh, pz, 0.0), axis=1, keepdims=True)
        sidx = jnp.where(iota_m == i, nxt, sidx)
        dx = px - cx
        dy = py - cy
        dz = pz - cz
        d = dx * dx + dy * dy + dz * dz
        return (jnp.minimum(dists, d), sidx)

    _, sidx = lax.fori_loop(1, _m, body, state0)
    idx_ref[...] = sidx


_fps_call = pl.pallas_call(
    _fps_body,
    out_shape=jax.ShapeDtypeStruct((_B, _m), jnp.int32),
)


def _sc_body(posx_h, posy_h, posz_h, xx_h, xy_h, xz_h,
             seli_h,
             feat_h, selpx_h, selpy_h, selpz_h,
             px_v, py_v, pz_v, xv0, xv1, xv2,
             sci, spb,
             cand_i, fin_i, stage0, stage1, sem0, sem1):
    wid = lax.axis_index("s") * _NC + lax.axis_index("c")
    cloud = wid // 2
    half = wid % 2
    row0 = cloud * _m + half * _HM

    pltpu.sync_copy(posx_h.at[cloud], px_v)
    pltpu.sync_copy(posy_h.at[cloud], py_v)
    pltpu.sync_copy(posz_h.at[cloud], pz_v)
    pltpu.sync_copy(xx_h.at[cloud], xv0)
    pltpu.sync_copy(xy_h.at[cloud], xv1)
    pltpu.sync_copy(xz_h.at[cloud], xv2)
    sl_half = pl.ds(half * _HM, _HM)
    pltpu.sync_copy(seli_h.at[cloud, sl_half], sci)

    for stage in (stage0, stage1):
        for g in range(_K * _F // 16):
            stage[pl.ds(g * 16, 16)] = jnp.zeros((16,), jnp.float32)

    lane16 = lax.iota(jnp.int32, 16)
    r2 = jnp.float32(_R2)

    def cand_d2(c, cnt, cx, cy, cz):
        sl = pl.ds(c * 16, 16)
        iv = cand_i[sl]
        dx = plsc.load_gather(px_v, [iv]) - cx
        dy = plsc.load_gather(py_v, [iv]) - cy
        dz = plsc.load_gather(pz_v, [iv]) - cz
        d2 = dx * dx + dy * dy + dz * dz
        ln = c * 16 + lane16
        return plsc.bitcast(d2, jnp.int32), iv, ln < cnt

    def do_row(r, stage, sem, other_stage, other_sem, first):
        rsplat = jnp.full((16,), r, jnp.int32)
        si = plsc.load_gather(sci, [rsplat])
        cx = plsc.load_gather(px_v, [si])
        cy = plsc.load_gather(py_v, [si])
        cz = plsc.load_gather(pz_v, [si])

        def chunk4(c4, cur):
            part = []
            for u in range(4):
                c = c4 * 4 + u
                sl = pl.ds(c * 16, 16)
                dx = px_v[sl] - cx
                dy = py_v[sl] - cy
                dz = pz_v[sl] - cz
                d2 = dx * dx + dy * dy + dz * dz
                msk = d2 <= r2
                pc = plsc.all_reduce_population_count(msk)
                part.append((c * 16 + lane16, msk, pc))
            s = cur
            for idxv, msk, pc in part:
                plsc.store_compressed(cand_i.at[pl.ds(s, 16)], idxv, mask=msk)
                s = s + pc[0]
            return s

        cnt = lax.fori_loop(0, _n // 64, chunk4, jnp.int32(0))

        @pl.when(cnt > _K)
        def _fixup():
            nch = (cnt + 15) // 16

            def count_le(v):
                def cc(c, acc):
                    dv, _, inb = cand_d2(c, cnt, cx, cy, cz)
                    mk = (dv <= v) & inb
                    return acc + plsc.all_reduce_population_count(mk)[0]
                return lax.fori_loop(0, nch, cc, jnp.int32(0))

            def bcond(state):
                lo, hi = state
                return lo < hi

            def bstep(state):
                lo, hi = state
                mid = (lo + hi) // 2
                big = count_le(mid) >= _K
                return (jnp.where(big, lo, mid + 1), jnp.where(big, mid, hi))

            vk, _ = lax.while_loop(
                bcond, bstep, (jnp.int32(0), jnp.int32(_R2_BITS)))

            def sel1(c, cur):
                dv, iv, inb = cand_d2(c, cnt, cx, cy, cz)
                mk = (dv < vk) & inb
                plsc.store_compressed(fin_i.at[pl.ds(cur, 16)], iv, mask=mk)
                return cur + plsc.all_reduce_population_count(mk)[0]

            cur = lax.fori_loop(0, nch, sel1, jnp.int32(0))

            def sel2(c, cur):
                dv, iv, inb = cand_d2(c, cnt, cx, cy, cz)
                mk = (dv == vk) & inb
                pref = plsc.cumsum(mk.astype(jnp.int32))
                keep = mk & ((cur + pref) <= _K)
                plsc.store_compressed(fin_i.at[pl.ds(cur, 16)], iv, mask=keep)
                return cur + plsc.all_reduce_population_count(keep)[0]

            lax.fori_loop(0, nch, sel2, cur)
            for g in range(_K // 16):
                sl = pl.ds(g * 16, 16)
                cand_i[sl] = fin_i[sl]

        @pl.when(jnp.logical_not(first))
        def _drain():
            pltpu.make_async_copy(stage, feat_h.at[row0], sem).wait()

        cnt2 = jnp.minimum(cnt, _K)
        for g in range(_K // 16):
            ln = g * 16 + lane16
            cv = cand_i[pl.ds(g * 16, 16)]
            jv = jnp.where(ln < cnt2, cv, si)
            f0 = plsc.load_gather(xv0, [jv])
            f1 = plsc.load_gather(xv1, [jv])
            f2 = plsc.load_gather(xv2, [jv])
            f3 = plsc.load_gather(px_v, [jv]) - cx
            f4 = plsc.load_gather(py_v, [jv]) - cy
            f5 = plsc.load_gather(pz_v, [jv]) - cz
            base = ln * _F
            plsc.store_scatter(stage, [base + 0], f0)
            plsc.store_scatter(stage, [base + 1], f1)
            plsc.store_scatter(stage, [base + 2], f2)
            plsc.store_scatter(stage, [base + 3], f3)
            plsc.store_scatter(stage, [base + 4], f4)
            plsc.store_scatter(stage, [base + 5], f5)
        pltpu.async_copy(stage, feat_h.at[row0 + r], sem)

    def row_pair(rr, carry):
        do_row(rr * 2, stage0, sem0, stage1, sem1, rr == 0)
        do_row(rr * 2 + 1, stage1, sem1, stage0, sem0, rr == 0)
        return carry

    lax.fori_loop(0, _HM // 2, row_pair, jnp.int32(0))
    for g in range(_HM // 16):
        iv = sci[pl.ds(g * 16, 16)]
        spb[pl.ds(g * 16, 16)] = plsc.load_gather(px_v, [iv])
        spb[pl.ds(_HM + g * 16, 16)] = plsc.load_gather(py_v, [iv])
        spb[pl.ds(2 * _HM + g * 16, 16)] = plsc.load_gather(pz_v, [iv])
    for ch, out_h in enumerate((selpx_h, selpy_h, selpz_h)):
        pltpu.sync_copy(spb.at[pl.ds(ch * _HM, _HM)],
                        out_h.at[pl.ds(row0, _HM)])
    pltpu.make_async_copy(stage0, feat_h.at[row0], sem0).wait()
    pltpu.make_async_copy(stage1, feat_h.at[row0], sem1).wait()


@functools.cache
def _make_sc_call():
    mesh = plsc.VectorSubcoreMesh(core_axis_name="c", subcore_axis_name="s")
    return functools.partial(
        pl.kernel,
        mesh=mesh,
        compiler_params=pltpu.CompilerParams(needs_layout_passes=False),
        out_type=[jax.ShapeDtypeStruct((_B * _m, _K * _F), jnp.float32),
                  jax.ShapeDtypeStruct((_B * _m,), jnp.float32),
                  jax.ShapeDtypeStruct((_B * _m,), jnp.float32),
                  jax.ShapeDtypeStruct((_B * _m,), jnp.float32)],
        scratch_types=[
            pltpu.VMEM((_n,), jnp.float32),
            pltpu.VMEM((_n,), jnp.float32),
            pltpu.VMEM((_n,), jnp.float32),
            pltpu.VMEM((_n,), jnp.float32),
            pltpu.VMEM((_n,), jnp.float32),
            pltpu.VMEM((_n,), jnp.float32),
            pltpu.VMEM((_HM,), jnp.int32),
            pltpu.VMEM((3 * _HM,), jnp.float32),
            pltpu.VMEM((_n + 32,), jnp.int32),
            pltpu.VMEM((_K + 16,), jnp.int32),
            pltpu.VMEM((_K * _F,), jnp.float32),
            pltpu.VMEM((_K * _F,), jnp.float32),
            pltpu.SemaphoreType.DMA,
            pltpu.SemaphoreType.DMA,
        ],
    )(_sc_body)


_EB = 8192


def _mlp_body(f_ref, w1_ref, b1_ref, w2_ref, b2_ref, w3_ref, b3_ref, o_ref):
    f = f_ref[...]
    h = jnp.dot(f, w1_ref[...], preferred_element_type=jnp.float32)
    h = jnp.maximum(h + b1_ref[...], 0.0)
    h = jnp.dot(h, w2_ref[...], preferred_element_type=jnp.float32)
    h = jnp.maximum(h + b2_ref[...], 0.0)
    h = jnp.dot(h, w3_ref[...], preferred_element_type=jnp.float32)
    h = jnp.maximum(h + b3_ref[...], 0.0)
    h = h.reshape(_EB // _K, _K, _OUT)
    o_ref[...] = jnp.max(h, axis=1)


_mlp_call = pl.pallas_call(
    _mlp_body,
    grid=(_B * _m * _K // _EB,),
    in_specs=[
        pl.BlockSpec((_EB, _F), lambda i: (i, 0)),
        pl.BlockSpec((_F, _H), lambda i: (0, 0)),
        pl.BlockSpec((1, _H), lambda i: (0, 0)),
        pl.BlockSpec((_H, _H), lambda i: (0, 0)),
        pl.BlockSpec((1, _H), lambda i: (0, 0)),
        pl.BlockSpec((_H, _OUT), lambda i: (0, 0)),
        pl.BlockSpec((1, _OUT), lambda i: (0, 0)),
    ],
    out_specs=pl.BlockSpec((_EB // _K, _OUT), lambda i: (i, 0)),
    out_shape=jax.ShapeDtypeStruct((_B * _m, _OUT), jnp.float32),
)


def kernel(x, pos, batch, W1, b1, W2, b2, W3, b3):
    posb = pos.reshape(_B, _n, 3)
    xb = x.reshape(_B, _n, 3)
    posx = posb[:, :, 0]
    posy = posb[:, :, 1]
    posz = posb[:, :, 2]
    xx = xb[:, :, 0]
    xy = xb[:, :, 1]
    xz = xb[:, :, 2]

    sel_idx = _fps_call(posx, posy, posz)

    feat, spx, spy, spz = _make_sc_call()(posx, posy, posz, xx, xy, xz, sel_idx)
    featr = feat.reshape(_B * _m * _K, _F)

    W1p = jnp.concatenate([W1, jnp.zeros((_F - W1.shape[0], _H), jnp.float32)], axis=0)
    out = _mlp_call(featr, W1p, b1.reshape(1, _H), W2, b2.reshape(1, _H),
                    W3, b3.reshape(1, _OUT))

    sel_pos = jnp.stack([spx, spy, spz], axis=-1)
    sel_batch = jnp.repeat(jnp.arange(_B, dtype=batch.dtype), _m)
    return out, sel_pos, sel_batch


# speedup vs baseline: 1.5580x; 1.1050x over previous
import functools

import numpy as np
import jax
import jax.numpy as jnp
from jax import lax
from jax.experimental import pallas as pl
from jax.experimental.pallas import tpu as pltpu
from jax.experimental.pallas import tpu_sc as plsc

_B = 16
_n = 2048
_m = 512
_K = 64
_H = 64
_OUT = 64
_F = 8
_R2 = np.float32(0.15 * 0.15)
_R2_BITS = int(np.array(0.15 * 0.15, np.float32).view(np.int32))

_NC = 2
_NS = 16
_NW = _NC * _NS
_HM = (_B * _m) // _NW


def _fps_body(px_ref, py_ref, pz_ref, idx_ref):
    px = px_ref[...]
    py = py_ref[...]
    pz = pz_ref[...]
    iota = lax.broadcasted_iota(jnp.int32, (_B, _n), 1)
    iota_m = lax.broadcasted_iota(jnp.int32, (_B, _m), 1)
    cx = px[:, 0:1]
    cy = py[:, 0:1]
    cz = pz[:, 0:1]
    dx = px - cx
    dy = py - cy
    dz = pz - cz
    dists0 = dx * dx + dy * dy + dz * dz
    state0 = (dists0, jnp.zeros((_B, _m), jnp.int32))

    def body(i, state):
        dists, sidx = state
        maxv = jnp.max(dists, axis=1, keepdims=True)
        nxt = jnp.min(jnp.where(dists == maxv, iota, _n), axis=1, keepdims=True)
        oh = iota == nxt
        cx = jnp.sum(jnp.where(oh, px, 0.0), axis=1, keepdims=True)
        cy = jnp.sum(jnp.where(oh, py, 0.0), axis=1, keepdims=True)
        cz = jnp.sum(jnp.where(oh, pz, 0.0), axis=1, keepdims=True)
        sidx = jnp.where(iota_m == i, nxt, sidx)
        dx = px - cx
        dy = py - cy
        dz = pz - cz
        d = dx * dx + dy * dy + dz * dz
        return (jnp.minimum(dists, d), sidx)

    _, sidx = lax.fori_loop(1, _m, body, state0)
    idx_ref[...] = sidx


_fps_call = pl.pallas_call(
    _fps_body,
    out_shape=jax.ShapeDtypeStruct((_B, _m), jnp.int32),
)


def _sc_body(posx_h, posy_h, posz_h, xx_h, xy_h, xz_h,
             seli_h,
             feat_h, selpx_h, selpy_h, selpz_h,
             px_v, py_v, pz_v, xv0, xv1, xv2,
             sci, spb,
             cand_i, fin_i, stage0, stage1, sem0, sem1):
    wid = lax.axis_index("s") * _NC + lax.axis_index("c")
    cloud = wid // 2
    half = wid % 2
    row0 = cloud * _m + half * _HM

    pltpu.sync_copy(posx_h.at[cloud], px_v)
    pltpu.sync_copy(posy_h.at[cloud], py_v)
    pltpu.sync_copy(posz_h.at[cloud], pz_v)
    pltpu.sync_copy(xx_h.at[cloud], xv0)
    pltpu.sync_copy(xy_h.at[cloud], xv1)
    pltpu.sync_copy(xz_h.at[cloud], xv2)
    sl_half = pl.ds(half * _HM, _HM)
    pltpu.sync_copy(seli_h.at[cloud, sl_half], sci)

    for stage in (stage0, stage1):
        for g in range(_K * _F // 16):
            stage[pl.ds(g * 16, 16)] = jnp.zeros((16,), jnp.float32)

    lane16 = lax.iota(jnp.int32, 16)
    r2 = jnp.float32(_R2)

    def cand_d2(c, cnt, cx, cy, cz):
        sl = pl.ds(c * 16, 16)
        iv = cand_i[sl]
        dx = plsc.load_gather(px_v, [iv]) - cx
        dy = plsc.load_gather(py_v, [iv]) - cy
        dz = plsc.load_gather(pz_v, [iv]) - cz
        d2 = dx * dx + dy * dy + dz * dz
        ln = c * 16 + lane16
        return plsc.bitcast(d2, jnp.int32), iv, ln < cnt

    def do_row(r, stage, sem, other_stage, other_sem, first):
        rsplat = jnp.full((16,), r, jnp.int32)
        si = plsc.load_gather(sci, [rsplat])
        cx = plsc.load_gather(px_v, [si])
        cy = plsc.load_gather(py_v, [si])
        cz = plsc.load_gather(pz_v, [si])

        def chunk4(c4, cur):
            part = []
            for u in range(8):
                c = c4 * 8 + u
                sl = pl.ds(c * 16, 16)
                dx = px_v[sl] - cx
                dy = py_v[sl] - cy
                dz = pz_v[sl] - cz
                d2 = dx * dx + dy * dy + dz * dz
                msk = d2 <= r2
                pc = plsc.all_reduce_population_count(msk)
                part.append((c * 16 + lane16, msk, pc))
            s = cur
            for idxv, msk, pc in part:
                plsc.store_compressed(cand_i.at[pl.ds(s, 16)], idxv, mask=msk)
                s = s + pc[0]
            return s

        cnt = lax.fori_loop(0, _n // 128, chunk4, jnp.int32(0))

        @pl.when(cnt > _K)
        def _fixup():
            nch = (cnt + 15) // 16

            def count_le(v):
                def cc(c, acc):
                    dv, _, inb = cand_d2(c, cnt, cx, cy, cz)
                    mk = (dv <= v) & inb
                    return acc + plsc.all_reduce_population_count(mk)[0]
                return lax.fori_loop(0, nch, cc, jnp.int32(0))

            def bcond(state):
                lo, hi = state
                return lo < hi

            def bstep(state):
                lo, hi = state
                mid = (lo + hi) // 2
                big = count_le(mid) >= _K
                return (jnp.where(big, lo, mid + 1), jnp.where(big, mid, hi))

            vk, _ = lax.while_loop(
                bcond, bstep, (jnp.int32(0), jnp.int32(_R2_BITS)))

            def sel1(c, cur):
                dv, iv, inb = cand_d2(c, cnt, cx, cy, cz)
                mk = (dv < vk) & inb
                plsc.store_compressed(fin_i.at[pl.ds(cur, 16)], iv, mask=mk)
                return cur + plsc.all_reduce_population_count(mk)[0]

            cur = lax.fori_loop(0, nch, sel1, jnp.int32(0))

            def sel2(c, cur):
                dv, iv, inb = cand_d2(c, cnt, cx, cy, cz)
                mk = (dv == vk) & inb
                pref = plsc.cumsum(mk.astype(jnp.int32))
                keep = mk & ((cur + pref) <= _K)
                plsc.store_compressed(fin_i.at[pl.ds(cur, 16)], iv, mask=keep)
                return cur + plsc.all_reduce_population_count(keep)[0]

            lax.fori_loop(0, nch, sel2, cur)
            for g in range(_K // 16):
                sl = pl.ds(g * 16, 16)
                cand_i[sl] = fin_i[sl]

        @pl.when(jnp.logical_not(first))
        def _drain():
            pltpu.make_async_copy(stage, feat_h.at[row0], sem).wait()

        cnt2 = jnp.minimum(cnt, _K)
        for g in range(_K // 16):
            ln = g * 16 + lane16
            cv = cand_i[pl.ds(g * 16, 16)]
            jv = jnp.where(ln < cnt2, cv, si)
            f0 = plsc.load_gather(xv0, [jv])
            f1 = plsc.load_gather(xv1, [jv])
            f2 = plsc.load_gather(xv2, [jv])
            f3 = plsc.load_gather(px_v, [jv]) - cx
            f4 = plsc.load_gather(py_v, [jv]) - cy
            f5 = plsc.load_gather(pz_v, [jv]) - cz
            base = ln * _F
            plsc.store_scatter(stage, [base + 0], f0)
            plsc.store_scatter(stage, [base + 1], f1)
            plsc.store_scatter(stage, [base + 2], f2)
            plsc.store_scatter(stage, [base + 3], f3)
            plsc.store_scatter(stage, [base + 4], f4)
            plsc.store_scatter(stage, [base + 5], f5)
        pltpu.async_copy(stage, feat_h.at[row0 + r], sem)

    def row_pair(rr, carry):
        do_row(rr * 2, stage0, sem0, stage1, sem1, rr == 0)
        do_row(rr * 2 + 1, stage1, sem1, stage0, sem0, rr == 0)
        return carry

    lax.fori_loop(0, _HM // 2, row_pair, jnp.int32(0))
    for g in range(_HM // 16):
        iv = sci[pl.ds(g * 16, 16)]
        spb[pl.ds(g * 16, 16)] = plsc.load_gather(px_v, [iv])
        spb[pl.ds(_HM + g * 16, 16)] = plsc.load_gather(py_v, [iv])
        spb[pl.ds(2 * _HM + g * 16, 16)] = plsc.load_gather(pz_v, [iv])
    for ch, out_h in enumerate((selpx_h, selpy_h, selpz_h)):
        pltpu.sync_copy(spb.at[pl.ds(ch * _HM, _HM)],
                        out_h.at[pl.ds(row0, _HM)])
    pltpu.make_async_copy(stage0, feat_h.at[row0], sem0).wait()
    pltpu.make_async_copy(stage1, feat_h.at[row0], sem1).wait()


@functools.cache
def _make_sc_call():
    mesh = plsc.VectorSubcoreMesh(core_axis_name="c", subcore_axis_name="s")
    return functools.partial(
        pl.kernel,
        mesh=mesh,
        compiler_params=pltpu.CompilerParams(needs_layout_passes=False),
        out_type=[jax.ShapeDtypeStruct((_B * _m, _K * _F), jnp.float32),
                  jax.ShapeDtypeStruct((_B * _m,), jnp.float32),
                  jax.ShapeDtypeStruct((_B * _m,), jnp.float32),
                  jax.ShapeDtypeStruct((_B * _m,), jnp.float32)],
        scratch_types=[
            pltpu.VMEM((_n,), jnp.float32),
            pltpu.VMEM((_n,), jnp.float32),
            pltpu.VMEM((_n,), jnp.float32),
            pltpu.VMEM((_n,), jnp.float32),
            pltpu.VMEM((_n,), jnp.float32),
            pltpu.VMEM((_n,), jnp.float32),
            pltpu.VMEM((_HM,), jnp.int32),
            pltpu.VMEM((3 * _HM,), jnp.float32),
            pltpu.VMEM((_n + 32,), jnp.int32),
            pltpu.VMEM((_K + 16,), jnp.int32),
            pltpu.VMEM((_K * _F,), jnp.float32),
            pltpu.VMEM((_K * _F,), jnp.float32),
            pltpu.SemaphoreType.DMA,
            pltpu.SemaphoreType.DMA,
        ],
    )(_sc_body)


_EB = 8192


def _mlp_body(f_ref, w1_ref, b1_ref, w2_ref, b2_ref, w3_ref, b3_ref, o_ref):
    f = f_ref[...]
    h = jnp.dot(f, w1_ref[...], preferred_element_type=jnp.float32)
    h = jnp.maximum(h + b1_ref[...], 0.0)
    h = jnp.dot(h, w2_ref[...], preferred_element_type=jnp.float32)
    h = jnp.maximum(h + b2_ref[...], 0.0)
    h = jnp.dot(h, w3_ref[...], preferred_element_type=jnp.float32)
    h = jnp.maximum(h + b3_ref[...], 0.0)
    h = h.reshape(_EB // _K, _K, _OUT)
    o_ref[...] = jnp.max(h, axis=1)


_mlp_call = pl.pallas_call(
    _mlp_body,
    grid=(_B * _m * _K // _EB,),
    in_specs=[
        pl.BlockSpec((_EB, _F), lambda i: (i, 0)),
        pl.BlockSpec((_F, _H), lambda i: (0, 0)),
        pl.BlockSpec((1, _H), lambda i: (0, 0)),
        pl.BlockSpec((_H, _H), lambda i: (0, 0)),
        pl.BlockSpec((1, _H), lambda i: (0, 0)),
        pl.BlockSpec((_H, _OUT), lambda i: (0, 0)),
        pl.BlockSpec((1, _OUT), lambda i: (0, 0)),
    ],
    out_specs=pl.BlockSpec((_EB // _K, _OUT), lambda i: (i, 0)),
    out_shape=jax.ShapeDtypeStruct((_B * _m, _OUT), jnp.float32),
)


def kernel(x, pos, batch, W1, b1, W2, b2, W3, b3):
    posb = pos.reshape(_B, _n, 3)
    xb = x.reshape(_B, _n, 3)
    posx = posb[:, :, 0]
    posy = posb[:, :, 1]
    posz = posb[:, :, 2]
    xx = xb[:, :, 0]
    xy = xb[:, :, 1]
    xz = xb[:, :, 2]

    sel_idx = _fps_call(posx, posy, posz)

    feat, spx, spy, spz = _make_sc_call()(posx, posy, posz, xx, xy, xz, sel_idx)
    featr = feat.reshape(_B * _m * _K, _F)

    W1p = jnp.concatenate([W1, jnp.zeros((_F - W1.shape[0], _H), jnp.float32)], axis=0)
    out = _mlp_call(featr, W1p, b1.reshape(1, _H), W2, b2.reshape(1, _H),
                    W3, b3.reshape(1, _OUT))

    sel_pos = jnp.stack([spx, spy, spz], axis=-1)
    sel_batch = jnp.repeat(jnp.arange(_B, dtype=batch.dtype), _m)
    return out, sel_pos, sel_batch
